# baseline (speedup 1.0000x reference)
import math

import jax
import jax.numpy as jnp
from jax.experimental import pallas as pl
from jax.experimental.pallas import tpu as pltpu


def _pick_chunk(e):
    for c in range(min(2048, e), 0, -1):
        if e % c == 0:
            return c
    return e


def _edge_pass(edges_blk, feats, track_deg):
    nblk, _, chunk = edges_blk.shape
    n, f = feats.shape

    def body(*refs):
        if track_deg:
            e_ref, h_ref, agg_ref, deg_ref = refs
        else:
            e_ref, h_ref, agg_ref = refs
            deg_ref = None
        i = pl.program_id(0)

        @pl.when(i == 0)
        def _():
            agg_ref[...] = jnp.zeros_like(agg_ref)
            if deg_ref is not None:
                deg_ref[...] = jnp.zeros_like(deg_ref)

        def step(k, carry):
            s = e_ref[0, 0, k]
            d = e_ref[0, 1, k]
            agg_ref[pl.ds(d, 1), :] = agg_ref[pl.ds(d, 1), :] + h_ref[pl.ds(s, 1), :]
            if deg_ref is not None:
                deg_ref[pl.ds(d, 1), :] = deg_ref[pl.ds(d, 1), :] + 1.0
            return carry

        jax.lax.fori_loop(0, chunk, step, 0)

    out_shapes = [jax.ShapeDtypeStruct((n, f), jnp.float32)]
    out_specs = [pl.BlockSpec((n, f), lambda i: (0, 0))]
    if track_deg:
        out_shapes.append(jax.ShapeDtypeStruct((n, f), jnp.float32))
        out_specs.append(pl.BlockSpec((n, f), lambda i: (0, 0)))

    res = pl.pallas_call(
        body,
        grid=(nblk,),
        in_specs=[
            pl.BlockSpec((1, 2, chunk), lambda i: (i, 0, 0),
                         memory_space=pltpu.SMEM),
            pl.BlockSpec((n, f), lambda i: (0, 0)),
        ],
        out_specs=out_specs,
        out_shape=out_shapes,
    )(edges_blk, feats)
    return res


def _dense_layer(agg, deg, h_prev, w_rel, w_root, b):
    n, f = h_prev.shape

    def body(agg_ref, deg_ref, h_ref, wr_ref, wo_ref, b_ref, out_ref):
        aggm = agg_ref[...] / jnp.maximum(deg_ref[...], 1.0)
        out = (jnp.dot(aggm, wr_ref[...], preferred_element_type=jnp.float32)
               + jnp.dot(h_ref[...], wo_ref[...], preferred_element_type=jnp.float32)
               + b_ref[...])
        out_ref[...] = jnp.maximum(out, 0.0)

    return pl.pallas_call(
        body,
        out_shape=jax.ShapeDtypeStruct((n, w_rel.shape[1]), jnp.float32),
    )(agg, deg, h_prev, w_rel, w_root, b.reshape(1, -1))


def _dense_layer_with_u(agg, deg, h_prev, w_rel, w_root, b, wp_rel):
    n, f = h_prev.shape

    def body(agg_ref, deg_ref, h_ref, wr_ref, wo_ref, b_ref, wp_ref, h3_ref, ub_ref):
        aggm = agg_ref[...] / jnp.maximum(deg_ref[...], 1.0)
        h3 = (jnp.dot(aggm, wr_ref[...], preferred_element_type=jnp.float32)
              + jnp.dot(h_ref[...], wo_ref[...], preferred_element_type=jnp.float32)
              + b_ref[...])
        h3 = jnp.maximum(h3, 0.0)
        h3_ref[...] = h3
        u = (jnp.dot(h_ref[...], wp_ref[0:f, :], preferred_element_type=jnp.float32)
             + jnp.dot(h3, wp_ref[f:2 * f, :], preferred_element_type=jnp.float32))
        ub_ref[...] = jnp.broadcast_to(u, (n, f))

    return pl.pallas_call(
        body,
        out_shape=[
            jax.ShapeDtypeStruct((n, f), jnp.float32),
            jax.ShapeDtypeStruct((n, f), jnp.float32),
        ],
    )(agg, deg, h_prev, w_rel, w_root, b.reshape(1, -1), wp_rel)


def _final_stage(uagg, h2, h3, wp_root, bp, wl1, bl1, wl2, bl2, k):
    n, f = h2.shape
    kf = float(k)

    def body(uagg_ref, h2_ref, h3_ref, wpr_ref, bp_ref, wl1_ref, bl1_ref,
             wl2_ref, bl2_ref, out_ref):
        h2v = h2_ref[...]
        h3v = h3_ref[...]
        score = (uagg_ref[:, 0:1]
                 + jnp.dot(h2v, wpr_ref[0:f, :], preferred_element_type=jnp.float32)
                 + jnp.dot(h3v, wpr_ref[f:2 * f, :], preferred_element_type=jnp.float32)
                 + bp_ref[0, 0])

        smin = jnp.min(score)
        smax = jnp.max(score)
        hi0 = smax + jnp.abs(smax) * 1e-6 + 1e-6

        def bisect(_, lohi):
            lo, hi = lohi
            mid = lo + (hi - lo) * 0.5
            cnt = jnp.sum((score >= mid).astype(jnp.float32))
            take = cnt >= kf
            return (jnp.where(take, mid, lo), jnp.where(take, hi, mid))

        lo, hi = jax.lax.fori_loop(0, 200, bisect, (smin, hi0))

        sel_hi = score >= hi
        n_hi = jnp.sum(sel_hi.astype(jnp.float32))
        zone = (score >= lo) & (score < hi)
        r = kf - n_hi
        idx = jax.lax.broadcasted_iota(jnp.int32, (n, 1), 0)

        def bisect_idx(_, lohi):
            lo_m, hi_m = lohi
            mid = (lo_m + hi_m) // 2
            c = jnp.sum((zone & (idx < mid)).astype(jnp.float32))
            ge = c >= r
            return (jnp.where(ge, lo_m, mid), jnp.where(ge, mid, hi_m))

        _, m_star = jax.lax.fori_loop(0, 15, bisect_idx,
                                      (jnp.int32(0), jnp.int32(n)))
        sel = sel_hi | (zone & (idx < m_star))
        w = jnp.where(sel, jnp.tanh(score), 0.0)

        pooled2 = jnp.sum(h2v * w, axis=0, keepdims=True) / kf
        pooled3 = jnp.sum(h3v * w, axis=0, keepdims=True) / kf
        hid = (jnp.dot(pooled2, wl1_ref[0:f, :], preferred_element_type=jnp.float32)
               + jnp.dot(pooled3, wl1_ref[f:2 * f, :], preferred_element_type=jnp.float32)
               + bl1_ref[...])
        hid = jnp.maximum(hid, 0.0)
        out_ref[...] = (jnp.dot(hid, wl2_ref[...], preferred_element_type=jnp.float32)
                        + bl2_ref[...])

    return pl.pallas_call(
        body,
        out_shape=jax.ShapeDtypeStruct((1, 1), jnp.float32),
    )(uagg, h2, h3, wp_root, bp.reshape(1, 1), wl1, bl1.reshape(1, -1),
      wl2, bl2.reshape(1, 1))


def kernel(x, edge_index, batch, W1_rel, W1_root, b1, W2_rel, W2_root, b2,
           W3_rel, W3_root, b3, Wp_rel, Wp_root, bp, Wl1, bl1, Wl2, bl2):
    n = x.shape[0]
    e = edge_index.shape[1]
    k = int(math.ceil(0.5 * n))
    chunk = _pick_chunk(e)
    nblk = e // chunk

    edges = edge_index.astype(jnp.int32)
    edges_blk = jnp.transpose(edges.reshape(2, nblk, chunk), (1, 0, 2))

    agg1, deg = _edge_pass(edges_blk, x, track_deg=True)
    h1 = _dense_layer(agg1, deg, x, W1_rel, W1_root, b1)
    (agg2,) = _edge_pass(edges_blk, h1, track_deg=False)
    h2 = _dense_layer(agg2, deg, h1, W2_rel, W2_root, b2)
    (agg3,) = _edge_pass(edges_blk, h2, track_deg=False)
    h3, ub = _dense_layer_with_u(agg3, deg, h2, W3_rel, W3_root, b3, Wp_rel)
    (uagg,) = _edge_pass(edges_blk, ub, track_deg=False)
    return _final_stage(uagg, h2, h3, Wp_root, bp, Wl1, bl1, Wl2, bl2, k)


# speedup vs baseline: 1.9404x; 1.9404x over previous
import math

import jax
import jax.numpy as jnp
from jax.experimental import pallas as pl
from jax.experimental.pallas import tpu as pltpu


def _pick_chunk(e):
    for c in range(min(2048, e), 0, -1):
        if e % c == 0:
            return c
    return e


_LANES = 4


def _edge_pass(edges_blk, feats, track_deg):
    nblk, _, chunk = edges_blk.shape
    n, f = feats.shape
    assert chunk % _LANES == 0

    def body(*refs):
        e_ref, h_ref = refs[0], refs[1]
        if track_deg:
            agg_ref, deg_ref = refs[2], refs[3]
            accs = refs[4:4 + _LANES]
            degs = refs[4 + _LANES:]
        else:
            agg_ref, deg_ref = refs[2], None
            accs = refs[3:3 + _LANES]
            degs = ()
        i = pl.program_id(0)

        @pl.when(i == 0)
        def _():
            for a in accs:
                a[...] = jnp.zeros_like(a)
            for a in degs:
                a[...] = jnp.zeros_like(a)

        def step(k, carry):
            base = k * _LANES
            for j in range(_LANES):
                s = e_ref[0, 0, base + j]
                d = e_ref[0, 1, base + j]
                accs[j][pl.ds(d, 1), :] = (accs[j][pl.ds(d, 1), :]
                                           + h_ref[pl.ds(s, 1), :])
                if degs:
                    dg = degs[j % len(degs)]
                    dg[pl.ds(d, 1), :] = dg[pl.ds(d, 1), :] + 1.0
            return carry

        jax.lax.fori_loop(0, chunk // _LANES, step, 0)

        @pl.when(i == nblk - 1)
        def _():
            tot = accs[0][...]
            for a in accs[1:]:
                tot = tot + a[...]
            agg_ref[...] = tot
            if deg_ref is not None:
                dtot = degs[0][...]
                for a in degs[1:]:
                    dtot = dtot + a[...]
                deg_ref[...] = dtot

    out_shapes = [jax.ShapeDtypeStruct((n, f), jnp.float32)]
    out_specs = [pl.BlockSpec((n, f), lambda i: (0, 0))]
    scratch = [pltpu.VMEM((n, f), jnp.float32) for _ in range(_LANES)]
    if track_deg:
        out_shapes.append(jax.ShapeDtypeStruct((n, f), jnp.float32))
        out_specs.append(pl.BlockSpec((n, f), lambda i: (0, 0)))
        scratch += [pltpu.VMEM((n, f), jnp.float32) for _ in range(2)]

    res = pl.pallas_call(
        body,
        grid=(nblk,),
        in_specs=[
            pl.BlockSpec((1, 2, chunk), lambda i: (i, 0, 0),
                         memory_space=pltpu.SMEM),
            pl.BlockSpec((n, f), lambda i: (0, 0)),
        ],
        out_specs=out_specs,
        out_shape=out_shapes,
        scratch_shapes=scratch,
    )(edges_blk, feats)
    return res


def _dense_layer(agg, deg, h_prev, w_rel, w_root, b):
    n, f = h_prev.shape

    def body(agg_ref, deg_ref, h_ref, wr_ref, wo_ref, b_ref, out_ref):
        aggm = agg_ref[...] / jnp.maximum(deg_ref[...], 1.0)
        out = (jnp.dot(aggm, wr_ref[...], preferred_element_type=jnp.float32)
               + jnp.dot(h_ref[...], wo_ref[...], preferred_element_type=jnp.float32)
               + b_ref[...])
        out_ref[...] = jnp.maximum(out, 0.0)

    return pl.pallas_call(
        body,
        out_shape=jax.ShapeDtypeStruct((n, w_rel.shape[1]), jnp.float32),
    )(agg, deg, h_prev, w_rel, w_root, b.reshape(1, -1))


def _dense_layer_with_u(agg, deg, h_prev, w_rel, w_root, b, wp_rel):
    n, f = h_prev.shape

    def body(agg_ref, deg_ref, h_ref, wr_ref, wo_ref, b_ref, wp_ref, h3_ref, ub_ref):
        aggm = agg_ref[...] / jnp.maximum(deg_ref[...], 1.0)
        h3 = (jnp.dot(aggm, wr_ref[...], preferred_element_type=jnp.float32)
              + jnp.dot(h_ref[...], wo_ref[...], preferred_element_type=jnp.float32)
              + b_ref[...])
        h3 = jnp.maximum(h3, 0.0)
        h3_ref[...] = h3
        u = (jnp.dot(h_ref[...], wp_ref[0:f, :], preferred_element_type=jnp.float32)
             + jnp.dot(h3, wp_ref[f:2 * f, :], preferred_element_type=jnp.float32))
        ub_ref[...] = jnp.broadcast_to(u, (n, f))

    return pl.pallas_call(
        body,
        out_shape=[
            jax.ShapeDtypeStruct((n, f), jnp.float32),
            jax.ShapeDtypeStruct((n, f), jnp.float32),
        ],
    )(agg, deg, h_prev, w_rel, w_root, b.reshape(1, -1), wp_rel)


def _final_stage(uagg, h2, h3, wp_root, bp, wl1, bl1, wl2, bl2, k):
    n, f = h2.shape
    kf = float(k)

    def body(uagg_ref, h2_ref, h3_ref, wpr_ref, bp_ref, wl1_ref, bl1_ref,
             wl2_ref, bl2_ref, out_ref):
        h2v = h2_ref[...]
        h3v = h3_ref[...]
        score = (uagg_ref[:, 0:1]
                 + jnp.dot(h2v, wpr_ref[0:f, :], preferred_element_type=jnp.float32)
                 + jnp.dot(h3v, wpr_ref[f:2 * f, :], preferred_element_type=jnp.float32)
                 + bp_ref[0, 0])

        smin = jnp.min(score)
        smax = jnp.max(score)
        hi0 = smax + jnp.abs(smax) * 1e-6 + 1e-6

        def bisect(_, lohi):
            lo, hi = lohi
            mid = lo + (hi - lo) * 0.5
            cnt = jnp.sum((score >= mid).astype(jnp.float32))
            take = cnt >= kf
            return (jnp.where(take, mid, lo), jnp.where(take, hi, mid))

        lo, hi = jax.lax.fori_loop(0, 200, bisect, (smin, hi0))

        sel_hi = score >= hi
        n_hi = jnp.sum(sel_hi.astype(jnp.float32))
        zone = (score >= lo) & (score < hi)
        r = kf - n_hi
        idx = jax.lax.broadcasted_iota(jnp.int32, (n, 1), 0)

        def bisect_idx(_, lohi):
            lo_m, hi_m = lohi
            mid = (lo_m + hi_m) // 2
            c = jnp.sum((zone & (idx < mid)).astype(jnp.float32))
            ge = c >= r
            return (jnp.where(ge, lo_m, mid), jnp.where(ge, mid, hi_m))

        _, m_star = jax.lax.fori_loop(0, 15, bisect_idx,
                                      (jnp.int32(0), jnp.int32(n)))
        sel = sel_hi | (zone & (idx < m_star))
        w = jnp.where(sel, jnp.tanh(score), 0.0)

        pooled2 = jnp.sum(h2v * w, axis=0, keepdims=True) / kf
        pooled3 = jnp.sum(h3v * w, axis=0, keepdims=True) / kf
        hid = (jnp.dot(pooled2, wl1_ref[0:f, :], preferred_element_type=jnp.float32)
               + jnp.dot(pooled3, wl1_ref[f:2 * f, :], preferred_element_type=jnp.float32)
               + bl1_ref[...])
        hid = jnp.maximum(hid, 0.0)
        out_ref[...] = (jnp.dot(hid, wl2_ref[...], preferred_element_type=jnp.float32)
                        + bl2_ref[...])

    return pl.pallas_call(
        body,
        out_shape=jax.ShapeDtypeStruct((1, 1), jnp.float32),
    )(uagg, h2, h3, wp_root, bp.reshape(1, 1), wl1, bl1.reshape(1, -1),
      wl2, bl2.reshape(1, 1))


def kernel(x, edge_index, batch, W1_rel, W1_root, b1, W2_rel, W2_root, b2,
           W3_rel, W3_root, b3, Wp_rel, Wp_root, bp, Wl1, bl1, Wl2, bl2):
    n = x.shape[0]
    e = edge_index.shape[1]
    k = int(math.ceil(0.5 * n))
    chunk = _pick_chunk(e)
    nblk = e // chunk

    edges = edge_index.astype(jnp.int32)
    edges_blk = jnp.transpose(edges.reshape(2, nblk, chunk), (1, 0, 2))

    agg1, deg = _edge_pass(edges_blk, x, track_deg=True)
    h1 = _dense_layer(agg1, deg, x, W1_rel, W1_root, b1)
    (agg2,) = _edge_pass(edges_blk, h1, track_deg=False)
    h2 = _dense_layer(agg2, deg, h1, W2_rel, W2_root, b2)
    (agg3,) = _edge_pass(edges_blk, h2, track_deg=False)
    h3, ub = _dense_layer_with_u(agg3, deg, h2, W3_rel, W3_root, b3, Wp_rel)
    (uagg,) = _edge_pass(edges_blk, ub, track_deg=False)
    return _final_stage(uagg, h2, h3, Wp_root, bp, Wl1, bl1, Wl2, bl2, k)


# speedup vs baseline: 2.1717x; 1.1192x over previous
import math

import jax
import jax.numpy as jnp
from jax.experimental import pallas as pl
from jax.experimental.pallas import tpu as pltpu


def _pick_chunk(e):
    for c in range(min(2048, e), 0, -1):
        if e % c == 0:
            return c
    return e


def _edge_pass(edges_blk, feats, track_deg):
    nblk, _, chunk = edges_blk.shape
    n, f = feats.shape
    lanes = 4 if track_deg else 8
    assert chunk % lanes == 0

    def body(*refs):
        e_ref, h_ref = refs[0], refs[1]
        if track_deg:
            agg_ref, deg_ref = refs[2], refs[3]
            accs = refs[4:4 + lanes]
            degs = refs[4 + lanes:]
        else:
            agg_ref, deg_ref = refs[2], None
            accs = refs[3:3 + lanes]
            degs = ()
        i = pl.program_id(0)

        @pl.when(i == 0)
        def _():
            for a in accs:
                a[...] = jnp.zeros_like(a)
            for a in degs:
                a[...] = jnp.zeros_like(a)

        def step(k, carry):
            base = k * lanes
            for j in range(lanes):
                s = e_ref[0, 0, base + j]
                d = e_ref[0, 1, base + j]
                accs[j][pl.ds(d, 1), :] = (accs[j][pl.ds(d, 1), :]
                                           + h_ref[pl.ds(s, 1), :])
                if degs:
                    dg = degs[j % len(degs)]
                    dg[pl.ds(d, 1), :] = dg[pl.ds(d, 1), :] + 1.0
            return carry

        jax.lax.fori_loop(0, chunk // lanes, step, 0)

        @pl.when(i == nblk - 1)
        def _():
            tot = accs[0][...]
            for a in accs[1:]:
                tot = tot + a[...]
            agg_ref[...] = tot
            if deg_ref is not None:
                dtot = degs[0][...]
                for a in degs[1:]:
                    dtot = dtot + a[...]
                deg_ref[...] = dtot

    out_shapes = [jax.ShapeDtypeStruct((n, f), jnp.float32)]
    out_specs = [pl.BlockSpec((n, f), lambda i: (0, 0))]
    scratch = [pltpu.VMEM((n, f), jnp.float32) for _ in range(lanes)]
    if track_deg:
        out_shapes.append(jax.ShapeDtypeStruct((n, f), jnp.float32))
        out_specs.append(pl.BlockSpec((n, f), lambda i: (0, 0)))
        scratch += [pltpu.VMEM((n, f), jnp.float32) for _ in range(2)]

    res = pl.pallas_call(
        body,
        grid=(nblk,),
        in_specs=[
            pl.BlockSpec((1, 2, chunk), lambda i: (i, 0, 0),
                         memory_space=pltpu.SMEM),
            pl.BlockSpec((n, f), lambda i: (0, 0)),
        ],
        out_specs=out_specs,
        out_shape=out_shapes,
        scratch_shapes=scratch,
    )(edges_blk, feats)
    return res


def _dense_layer(agg, deg, h_prev, w_rel, w_root, b):
    n, f = h_prev.shape

    def body(agg_ref, deg_ref, h_ref, wr_ref, wo_ref, b_ref, out_ref):
        aggm = agg_ref[...] / jnp.maximum(deg_ref[...], 1.0)
        out = (jnp.dot(aggm, wr_ref[...], preferred_element_type=jnp.float32)
               + jnp.dot(h_ref[...], wo_ref[...], preferred_element_type=jnp.float32)
               + b_ref[...])
        out_ref[...] = jnp.maximum(out, 0.0)

    return pl.pallas_call(
        body,
        out_shape=jax.ShapeDtypeStruct((n, w_rel.shape[1]), jnp.float32),
    )(agg, deg, h_prev, w_rel, w_root, b.reshape(1, -1))


def _dense_layer_with_u(agg, deg, h_prev, w_rel, w_root, b, wp_rel):
    n, f = h_prev.shape

    def body(agg_ref, deg_ref, h_ref, wr_ref, wo_ref, b_ref, wp_ref, h3_ref, ub_ref):
        aggm = agg_ref[...] / jnp.maximum(deg_ref[...], 1.0)
        h3 = (jnp.dot(aggm, wr_ref[...], preferred_element_type=jnp.float32)
              + jnp.dot(h_ref[...], wo_ref[...], preferred_element_type=jnp.float32)
              + b_ref[...])
        h3 = jnp.maximum(h3, 0.0)
        h3_ref[...] = h3
        u = (jnp.dot(h_ref[...], wp_ref[0:f, :], preferred_element_type=jnp.float32)
             + jnp.dot(h3, wp_ref[f:2 * f, :], preferred_element_type=jnp.float32))
        ub_ref[...] = jnp.broadcast_to(u, (n, f))

    return pl.pallas_call(
        body,
        out_shape=[
            jax.ShapeDtypeStruct((n, f), jnp.float32),
            jax.ShapeDtypeStruct((n, f), jnp.float32),
        ],
    )(agg, deg, h_prev, w_rel, w_root, b.reshape(1, -1), wp_rel)


def _final_stage(uagg, h2, h3, wp_root, bp, wl1, bl1, wl2, bl2, k):
    n, f = h2.shape
    kf = float(k)

    def body(uagg_ref, h2_ref, h3_ref, wpr_ref, bp_ref, wl1_ref, bl1_ref,
             wl2_ref, bl2_ref, out_ref):
        h2v = h2_ref[...]
        h3v = h3_ref[...]
        score = (uagg_ref[:, 0:1]
                 + jnp.dot(h2v, wpr_ref[0:f, :], preferred_element_type=jnp.float32)
                 + jnp.dot(h3v, wpr_ref[f:2 * f, :], preferred_element_type=jnp.float32)
                 + bp_ref[0, 0])

        smin = jnp.min(score)
        smax = jnp.max(score)
        hi0 = smax + jnp.abs(smax) * 1e-6 + 1e-6

        def bisect(_, lohi):
            lo, hi = lohi
            mid = lo + (hi - lo) * 0.5
            cnt = jnp.sum((score >= mid).astype(jnp.float32))
            take = cnt >= kf
            return (jnp.where(take, mid, lo), jnp.where(take, hi, mid))

        lo, hi = jax.lax.fori_loop(0, 200, bisect, (smin, hi0))

        sel_hi = score >= hi
        n_hi = jnp.sum(sel_hi.astype(jnp.float32))
        zone = (score >= lo) & (score < hi)
        r = kf - n_hi
        idx = jax.lax.broadcasted_iota(jnp.int32, (n, 1), 0)

        def bisect_idx(_, lohi):
            lo_m, hi_m = lohi
            mid = (lo_m + hi_m) // 2
            c = jnp.sum((zone & (idx < mid)).astype(jnp.float32))
            ge = c >= r
            return (jnp.where(ge, lo_m, mid), jnp.where(ge, mid, hi_m))

        _, m_star = jax.lax.fori_loop(0, 15, bisect_idx,
                                      (jnp.int32(0), jnp.int32(n)))
        sel = sel_hi | (zone & (idx < m_star))
        w = jnp.where(sel, jnp.tanh(score), 0.0)

        pooled2 = jnp.sum(h2v * w, axis=0, keepdims=True) / kf
        pooled3 = jnp.sum(h3v * w, axis=0, keepdims=True) / kf
        hid = (jnp.dot(pooled2, wl1_ref[0:f, :], preferred_element_type=jnp.float32)
               + jnp.dot(pooled3, wl1_ref[f:2 * f, :], preferred_element_type=jnp.float32)
               + bl1_ref[...])
        hid = jnp.maximum(hid, 0.0)
        out_ref[...] = (jnp.dot(hid, wl2_ref[...], preferred_element_type=jnp.float32)
                        + bl2_ref[...])

    return pl.pallas_call(
        body,
        out_shape=jax.ShapeDtypeStruct((1, 1), jnp.float32),
    )(uagg, h2, h3, wp_root, bp.reshape(1, 1), wl1, bl1.reshape(1, -1),
      wl2, bl2.reshape(1, 1))


def kernel(x, edge_index, batch, W1_rel, W1_root, b1, W2_rel, W2_root, b2,
           W3_rel, W3_root, b3, Wp_rel, Wp_root, bp, Wl1, bl1, Wl2, bl2):
    n = x.shape[0]
    e = edge_index.shape[1]
    k = int(math.ceil(0.5 * n))
    chunk = _pick_chunk(e)
    nblk = e // chunk

    edges = edge_index.astype(jnp.int32)
    edges_blk = jnp.transpose(edges.reshape(2, nblk, chunk), (1, 0, 2))

    agg1, deg = _edge_pass(edges_blk, x, track_deg=True)
    h1 = _dense_layer(agg1, deg, x, W1_rel, W1_root, b1)
    (agg2,) = _edge_pass(edges_blk, h1, track_deg=False)
    h2 = _dense_layer(agg2, deg, h1, W2_rel, W2_root, b2)
    (agg3,) = _edge_pass(edges_blk, h2, track_deg=False)
    h3, ub = _dense_layer_with_u(agg3, deg, h2, W3_rel, W3_root, b3, Wp_rel)
    (uagg,) = _edge_pass(edges_blk, ub, track_deg=False)
    return _final_stage(uagg, h2, h3, Wp_root, bp, Wl1, bl1, Wl2, bl2, k)
